# baseline (speedup 1.0000x reference)
import functools
import jax
import jax.numpy as jnp
from jax import lax
from jax.experimental import pallas as pl
from jax.experimental.pallas import tpu as pltpu, tpu_sc as plsc

_B, _H, _W, _C = 2, 384, 384, 96
_RP = 16
_TCP = 176
_SCP = _H // 2 - _TCP
_NW = 32
_PPW = _B * _SCP // _NW
_QC = _C // 4

_mesh = plsc.VectorSubcoreMesh(core_axis_name="c", subcore_axis_name="s")


def _tc_body(x_ref, o_ref):
    xe = x_ref[0, :, 0]
    xo = x_ref[0, :, 1]
    w = xe.shape[2]

    even_w = (jax.lax.broadcasted_iota(jnp.int32, xe.shape, 2) & 1) == 0

    def pair_swap(a):
        return jnp.where(
            even_w,
            pltpu.roll(a, w - 1, axis=2),
            pltpu.roll(a, 1, axis=2),
        )

    pe = pair_swap(xe)
    po = pair_swap(xo)

    win_e = (xe > pe) | (even_w & (xe == pe))
    win_o = (xo > po) | (even_w & (xo == po))
    e_ge = jnp.maximum(xe, pe) >= jnp.maximum(xo, po)

    o_ref[0, :, 0] = jnp.where(win_e & e_ge, xe, 0.0)
    o_ref[0, :, 1] = jnp.where(win_o & ~e_ge, xo, 0.0)


@functools.partial(
    pl.kernel,
    mesh=_mesh,
    out_type=jax.ShapeDtypeStruct((_B * _SCP * 2, _C, _W), jnp.float32),
    compiler_params=pltpu.CompilerParams(needs_layout_passes=False),
    scratch_types=[
        pltpu.VMEM((_QC, _W), jnp.float32),
        pltpu.VMEM((_QC, _W), jnp.float32),
        pltpu.VMEM((_QC, _W), jnp.float32),
        pltpu.VMEM((_QC, _W), jnp.float32),
    ],
)
def _sc_unpool(x_hbm, o_hbm, xe_v, xo_v, oe_v, oo_v):
    wid = lax.axis_index("s") * 2 + lax.axis_index("c")
    lane = lax.iota(jnp.int32, 16)
    zero = jnp.zeros((16,), jnp.float32)

    def do_pair(j, _):
        p = wid * _PPW + j
        b = p // _SCP
        hp = _TCP + p % _SCP
        r0 = b * _H + 2 * hp
        s0 = 2 * p

        def do_chunk(q, _):
            c0 = q * _QC
            pltpu.sync_copy(x_hbm.at[r0, pl.ds(c0, _QC)], xe_v)
            pltpu.sync_copy(x_hbm.at[r0 + 1, pl.ds(c0, _QC)], xo_v)

            def do_row(r, _):
                rr = jnp.full((16,), r, jnp.int32)

                def do_grp(g, _):
                    ie = g * 32 + 2 * lane
                    io = ie + 1
                    a0 = plsc.load_gather(xe_v, [rr, ie])
                    a1 = plsc.load_gather(xe_v, [rr, io])
                    a2 = plsc.load_gather(xo_v, [rr, ie])
                    a3 = plsc.load_gather(xo_v, [rr, io])
                    w0 = a0 >= a1
                    w2 = a2 >= a3
                    ege = jnp.maximum(a0, a1) >= jnp.maximum(a2, a3)
                    plsc.store_scatter(oe_v, [rr, ie],
                                       jnp.where(w0 & ege, a0, zero))
                    plsc.store_scatter(oe_v, [rr, io],
                                       jnp.where(~w0 & ege, a1, zero))
                    plsc.store_scatter(oo_v, [rr, ie],
                                       jnp.where(w2 & ~ege, a2, zero))
                    plsc.store_scatter(oo_v, [rr, io],
                                       jnp.where(~w2 & ~ege, a3, zero))
                    return 0

                return lax.fori_loop(0, _W // 32, do_grp, 0)

            lax.fori_loop(0, _QC, do_row, 0)
            pltpu.sync_copy(oe_v, o_hbm.at[s0, pl.ds(c0, _QC)])
            pltpu.sync_copy(oo_v, o_hbm.at[s0 + 1, pl.ds(c0, _QC)])
            return 0

        return lax.fori_loop(0, 4, do_chunk, 0)

    lax.fori_loop(0, _PPW, do_pair, 0)


def kernel(x):
    xt = jnp.transpose(x, (0, 1, 3, 2))
    x5 = xt.reshape(_B, _H // 2, 2, _C, _W)

    sc_out = _sc_unpool(xt.reshape(_B * _H, _C, _W))

    tc_out = pl.pallas_call(
        _tc_body,
        grid=(_B, _TCP // _RP),
        in_specs=[pl.BlockSpec((1, _RP, 2, _C, _W), lambda b, i: (b, i, 0, 0, 0))],
        out_specs=pl.BlockSpec((1, _RP, 2, _C, _W), lambda b, i: (b, i, 0, 0, 0)),
        out_shape=jax.ShapeDtypeStruct((_B, _TCP, 2, _C, _W), x.dtype),
    )(x5)

    sc5 = sc_out.reshape(_B, _SCP, 2, _C, _W)

    out = jnp.concatenate([tc_out, sc5], axis=1)
    return jnp.transpose(out.reshape(_B, _H, _C, _W), (0, 1, 3, 2))


# speedup vs baseline: 1.0800x; 1.0800x over previous
import functools
import jax
import jax.numpy as jnp
from jax import lax
from jax.experimental import pallas as pl
from jax.experimental.pallas import tpu as pltpu, tpu_sc as plsc

_B, _H, _W, _C = 2, 384, 384, 96
_RP = 16
_TCP = 144
_SCP = _H // 2 - _TCP
_NW = 32
_PPW = _B * _SCP // _NW
_QC = _C // 4

_mesh = plsc.VectorSubcoreMesh(core_axis_name="c", subcore_axis_name="s")


def _tc_body(x_ref, o_ref):
    xe = x_ref[0, :, 0]
    xo = x_ref[0, :, 1]
    w = xe.shape[2]

    even_w = (jax.lax.broadcasted_iota(jnp.int32, xe.shape, 2) & 1) == 0

    def pair_swap(a):
        return jnp.where(
            even_w,
            pltpu.roll(a, w - 1, axis=2),
            pltpu.roll(a, 1, axis=2),
        )

    pe = pair_swap(xe)
    po = pair_swap(xo)

    win_e = (xe > pe) | (even_w & (xe == pe))
    win_o = (xo > po) | (even_w & (xo == po))
    e_ge = jnp.maximum(xe, pe) >= jnp.maximum(xo, po)

    o_ref[0, :, 0] = jnp.where(win_e & e_ge, xe, 0.0)
    o_ref[0, :, 1] = jnp.where(win_o & ~e_ge, xo, 0.0)


@functools.partial(
    pl.kernel,
    mesh=_mesh,
    out_type=jax.ShapeDtypeStruct((_B * _SCP * 2, _C, _W), jnp.float32),
    compiler_params=pltpu.CompilerParams(needs_layout_passes=False),
    scratch_types=[
        pltpu.VMEM((_QC, _W), jnp.float32),
        pltpu.VMEM((_QC, _W), jnp.float32),
        pltpu.VMEM((_QC, _W), jnp.float32),
        pltpu.VMEM((_QC, _W), jnp.float32),
    ],
)
def _sc_unpool(x_hbm, o_hbm, xe_v, xo_v, oe_v, oo_v):
    wid = lax.axis_index("s") * 2 + lax.axis_index("c")
    lane = lax.iota(jnp.int32, 16)
    zero = jnp.zeros((16,), jnp.float32)

    def do_pair(j, _):
        p = wid * _PPW + j
        b = p // _SCP
        hp = _TCP + p % _SCP
        r0 = b * _H + 2 * hp
        s0 = 2 * p

        def do_chunk(q, _):
            c0 = q * _QC
            pltpu.sync_copy(x_hbm.at[r0, pl.ds(c0, _QC)], xe_v)
            pltpu.sync_copy(x_hbm.at[r0 + 1, pl.ds(c0, _QC)], xo_v)

            def do_row(r, _):
                rr = jnp.full((16,), r, jnp.int32)

                def do_grp(g, _):
                    ie = g * 32 + 2 * lane
                    io = ie + 1
                    a0 = plsc.load_gather(xe_v, [rr, ie])
                    a1 = plsc.load_gather(xe_v, [rr, io])
                    a2 = plsc.load_gather(xo_v, [rr, ie])
                    a3 = plsc.load_gather(xo_v, [rr, io])
                    w0 = a0 >= a1
                    w2 = a2 >= a3
                    ege = jnp.maximum(a0, a1) >= jnp.maximum(a2, a3)
                    plsc.store_scatter(oe_v, [rr, ie],
                                       jnp.where(w0 & ege, a0, zero))
                    plsc.store_scatter(oe_v, [rr, io],
                                       jnp.where(~w0 & ege, a1, zero))
                    plsc.store_scatter(oo_v, [rr, ie],
                                       jnp.where(w2 & ~ege, a2, zero))
                    plsc.store_scatter(oo_v, [rr, io],
                                       jnp.where(~w2 & ~ege, a3, zero))
                    return 0

                return lax.fori_loop(0, _W // 32, do_grp, 0)

            lax.fori_loop(0, _QC, do_row, 0)
            pltpu.sync_copy(oe_v, o_hbm.at[s0, pl.ds(c0, _QC)])
            pltpu.sync_copy(oo_v, o_hbm.at[s0 + 1, pl.ds(c0, _QC)])
            return 0

        return lax.fori_loop(0, 4, do_chunk, 0)

    lax.fori_loop(0, _PPW, do_pair, 0)


def kernel(x):
    xt = jnp.transpose(x, (0, 1, 3, 2))
    x5 = xt.reshape(_B, _H // 2, 2, _C, _W)

    sc_out = _sc_unpool(xt.reshape(_B * _H, _C, _W))

    tc_out = pl.pallas_call(
        _tc_body,
        grid=(_B, _TCP // _RP),
        in_specs=[pl.BlockSpec((1, _RP, 2, _C, _W), lambda b, i: (b, i, 0, 0, 0))],
        out_specs=pl.BlockSpec((1, _RP, 2, _C, _W), lambda b, i: (b, i, 0, 0, 0)),
        out_shape=jax.ShapeDtypeStruct((_B, _TCP, 2, _C, _W), x.dtype),
    )(x5)

    sc5 = sc_out.reshape(_B, _SCP, 2, _C, _W)

    out = jnp.concatenate([tc_out, sc5], axis=1)
    return jnp.transpose(out.reshape(_B, _H, _C, _W), (0, 1, 3, 2))


# speedup vs baseline: 1.0881x; 1.0075x over previous
import functools
import jax
import jax.numpy as jnp
from jax import lax
from jax.experimental import pallas as pl
from jax.experimental.pallas import tpu as pltpu, tpu_sc as plsc

_B, _H, _W, _C = 2, 384, 384, 96
_RP = 16
_TCP = 144
_SCP = _H // 2 - _TCP
_NW = 32
_PPW = _B * _SCP // _NW
_QC = _C // 4

_mesh = plsc.VectorSubcoreMesh(core_axis_name="c", subcore_axis_name="s")


def _tc_body(x_ref, o_ref):
    xe = x_ref[0, :, 0]
    xo = x_ref[0, :, 1]
    w = xe.shape[2]

    even_w = (jax.lax.broadcasted_iota(jnp.int32, xe.shape, 2) & 1) == 0

    def pair_swap(a):
        return jnp.where(
            even_w,
            pltpu.roll(a, w - 1, axis=2),
            pltpu.roll(a, 1, axis=2),
        )

    pe = pair_swap(xe)
    po = pair_swap(xo)

    win_e = (xe > pe) | (even_w & (xe == pe))
    win_o = (xo > po) | (even_w & (xo == po))
    e_ge = jnp.maximum(xe, pe) >= jnp.maximum(xo, po)

    o_ref[0, :, 0] = jnp.where(win_e & e_ge, xe, 0.0)
    o_ref[0, :, 1] = jnp.where(win_o & ~e_ge, xo, 0.0)


@functools.partial(
    pl.kernel,
    mesh=_mesh,
    out_type=jax.ShapeDtypeStruct((_B * _SCP * 2, _C, _W), jnp.float32),
    compiler_params=pltpu.CompilerParams(needs_layout_passes=False),
    scratch_types=[
        pltpu.VMEM((_QC, _W), jnp.float32),
        pltpu.VMEM((_QC, _W), jnp.float32),
        pltpu.VMEM((_QC, _W), jnp.float32),
        pltpu.VMEM((_QC, _W), jnp.float32),
    ],
)
def _sc_unpool(x_hbm, o_hbm, xe_v, xo_v, oe_v, oo_v):
    wid = lax.axis_index("s") * 2 + lax.axis_index("c")
    lane = lax.iota(jnp.int32, 16)
    zero = jnp.zeros((16,), jnp.float32)

    def do_pair(j, _):
        p = wid * _PPW + j
        b = p // _SCP
        hp = _TCP + p % _SCP
        r0 = b * _H + 2 * hp
        s0 = 2 * p

        def do_chunk(q, _):
            c0 = q * _QC
            pltpu.sync_copy(x_hbm.at[r0, pl.ds(c0, _QC)], xe_v)
            pltpu.sync_copy(x_hbm.at[r0 + 1, pl.ds(c0, _QC)], xo_v)

            @plsc.parallel_loop(0, _QC * (_W // 32), unroll=4)
            def _grp_loop(t):
                r = t // (_W // 32)
                g = t % (_W // 32)
                rr = jnp.full((16,), r, jnp.int32)
                ie = g * 32 + 2 * lane
                io = ie + 1
                a0 = plsc.load_gather(xe_v, [rr, ie])
                a1 = plsc.load_gather(xe_v, [rr, io])
                a2 = plsc.load_gather(xo_v, [rr, ie])
                a3 = plsc.load_gather(xo_v, [rr, io])
                w0 = a0 >= a1
                w2 = a2 >= a3
                ege = jnp.maximum(a0, a1) >= jnp.maximum(a2, a3)
                plsc.store_scatter(oe_v, [rr, ie],
                                   jnp.where(w0 & ege, a0, zero))
                plsc.store_scatter(oe_v, [rr, io],
                                   jnp.where(~w0 & ege, a1, zero))
                plsc.store_scatter(oo_v, [rr, ie],
                                   jnp.where(w2 & ~ege, a2, zero))
                plsc.store_scatter(oo_v, [rr, io],
                                   jnp.where(~w2 & ~ege, a3, zero))
            pltpu.sync_copy(oe_v, o_hbm.at[s0, pl.ds(c0, _QC)])
            pltpu.sync_copy(oo_v, o_hbm.at[s0 + 1, pl.ds(c0, _QC)])
            return 0

        return lax.fori_loop(0, 4, do_chunk, 0)

    lax.fori_loop(0, _PPW, do_pair, 0)


def kernel(x):
    xt = jnp.transpose(x, (0, 1, 3, 2))
    x5 = xt.reshape(_B, _H // 2, 2, _C, _W)

    sc_out = _sc_unpool(xt.reshape(_B * _H, _C, _W))

    tc_out = pl.pallas_call(
        _tc_body,
        grid=(_B, _TCP // _RP),
        in_specs=[pl.BlockSpec((1, _RP, 2, _C, _W), lambda b, i: (b, i, 0, 0, 0))],
        out_specs=pl.BlockSpec((1, _RP, 2, _C, _W), lambda b, i: (b, i, 0, 0, 0)),
        out_shape=jax.ShapeDtypeStruct((_B, _TCP, 2, _C, _W), x.dtype),
    )(x5)

    sc5 = sc_out.reshape(_B, _SCP, 2, _C, _W)

    out = jnp.concatenate([tc_out, sc5], axis=1)
    return jnp.transpose(out.reshape(_B, _H, _C, _W), (0, 1, 3, 2))
